# baseline (speedup 1.0000x reference)
import jax
import jax.numpy as jnp
from jax import lax
from jax.experimental import pallas as pl
from jax.experimental.pallas import tpu as pltpu, tpu_sc as plsc

NUM_DOMAINS = 5
EMBED_DIM = 16
BATCH = 16384
L = 16

NC = 2
NS = 16
NW = NC * NS
B_PER_W = BATCH // NW
N_BLOCKS = B_PER_W // L
N_CHUNKS = 4
BLOCKS_PER_CHUNK = N_BLOCKS // N_CHUNKS
CHUNK_ELEMS = BLOCKS_PER_CHUNK * L * EMBED_DIM

_mesh = plsc.VectorSubcoreMesh(core_axis_name="c", subcore_axis_name="s")


def _body(ids_hbm, table_hbm, out_hbm, idx_v, tab_v, rows_v, sem_in, sem_out):
    wid = lax.axis_index("s") * NC + lax.axis_index("c")
    base = wid * B_PER_W
    ctab = pltpu.async_copy(table_hbm, tab_v, sem_in)
    cids = pltpu.async_copy(ids_hbm.at[pl.ds(base, B_PER_W)], idx_v, sem_in)
    ctab.wait()
    cids.wait()

    iota16 = lax.iota(jnp.int32, L) * EMBED_DIM
    out_copies = []
    for c in range(N_CHUNKS):
        for bb in range(BLOCKS_PER_CHUNK):
            b = c * BLOCKS_PER_CHUNK + bb
            v_src = idx_v[pl.ds(b * L, L)] * EMBED_DIM
            v_dst = iota16 + b * L * EMBED_DIM
            for j in range(EMBED_DIM):
                vals = plsc.load_gather(tab_v, [v_src + j])
                plsc.store_scatter(rows_v, [v_dst + j], vals)
        out_copies.append(
            pltpu.async_copy(
                rows_v.at[pl.ds(c * CHUNK_ELEMS, CHUNK_ELEMS)],
                out_hbm.at[pl.ds(base * EMBED_DIM + c * CHUNK_ELEMS, CHUNK_ELEMS)],
                sem_out,
            )
        )
    for cp in out_copies:
        cp.wait()


_sc_lookup = pl.kernel(
    _body,
    out_type=jax.ShapeDtypeStruct((BATCH * EMBED_DIM,), jnp.float32),
    mesh=_mesh,
    scratch_types=[
        pltpu.VMEM((B_PER_W,), jnp.int32),
        pltpu.VMEM((NUM_DOMAINS * EMBED_DIM,), jnp.float32),
        pltpu.VMEM((B_PER_W * EMBED_DIM,), jnp.float32),
        pltpu.SemaphoreType.DMA,
        pltpu.SemaphoreType.DMA,
    ],
    compiler_params=pltpu.CompilerParams(
        use_tc_tiling_on_sc=False, needs_layout_passes=False
    ),
)


@jax.jit
def kernel(domain_ids, table):
    flat = _sc_lookup(domain_ids.astype(jnp.int32), table.reshape(-1))
    return flat.reshape(BATCH, EMBED_DIM)


# speedup vs baseline: 1.0669x; 1.0669x over previous
import jax
import jax.numpy as jnp
from jax import lax
from jax.experimental import pallas as pl
from jax.experimental.pallas import tpu as pltpu, tpu_sc as plsc

NUM_DOMAINS = 5
EMBED_DIM = 16
BATCH = 16384
L = 16

NC = 2
NS = 16
NW = NC * NS
B_PER_W = BATCH // NW
N_BLOCKS = B_PER_W // L
N_CHUNKS = 4
BLOCKS_PER_CHUNK = N_BLOCKS // N_CHUNKS
CHUNK_ROWS = BLOCKS_PER_CHUNK * L

_mesh = plsc.VectorSubcoreMesh(core_axis_name="c", subcore_axis_name="s")


def _body(ids_hbm, table_hbm, out_hbm, idx_v, tab_v, rows_v, sem_in, sem_out):
    wid = lax.axis_index("s") * NC + lax.axis_index("c")
    base = wid * B_PER_W
    ctab = pltpu.async_copy(table_hbm, tab_v, sem_in)
    cids = pltpu.async_copy(ids_hbm.at[pl.ds(base, B_PER_W)], idx_v, sem_in)
    ctab.wait()
    cids.wait()

    iota = lax.iota(jnp.int32, L)
    out_copies = []
    for c in range(N_CHUNKS):

        def block(bb, _):
            b = c * BLOCKS_PER_CHUNK + bb
            v_ids = idx_v[pl.ds(b * L, L)]
            v_rows = iota + b * L
            for j in range(EMBED_DIM):
                col = jnp.full((L,), j, jnp.int32)
                vals = plsc.load_gather(tab_v, [v_ids, col])
                plsc.store_scatter(rows_v, [v_rows, col], vals)
            return 0

        lax.fori_loop(0, BLOCKS_PER_CHUNK, block, 0)
        out_copies.append(
            pltpu.async_copy(
                rows_v.at[pl.ds(c * CHUNK_ROWS, CHUNK_ROWS)],
                out_hbm.at[pl.ds(base + c * CHUNK_ROWS, CHUNK_ROWS)],
                sem_out,
            )
        )
    for cp in out_copies:
        cp.wait()


_sc_lookup = pl.kernel(
    _body,
    out_type=jax.ShapeDtypeStruct((BATCH, EMBED_DIM), jnp.float32),
    mesh=_mesh,
    scratch_types=[
        pltpu.VMEM((B_PER_W,), jnp.int32),
        pltpu.VMEM((NUM_DOMAINS, EMBED_DIM), jnp.float32),
        pltpu.VMEM((B_PER_W, EMBED_DIM), jnp.float32),
        pltpu.SemaphoreType.DMA,
        pltpu.SemaphoreType.DMA,
    ],
    compiler_params=pltpu.CompilerParams(
        use_tc_tiling_on_sc=False, needs_layout_passes=False
    ),
)


@jax.jit
def kernel(domain_ids, table):
    return _sc_lookup(domain_ids.astype(jnp.int32), table)
